# baseline (speedup 1.0000x reference)
import functools

import jax
import jax.numpy as jnp
from jax import lax
from jax.experimental import pallas as pl
from jax.experimental.pallas import tpu as pltpu
from jax.experimental.pallas import tpu_sc as plsc

_ODR = 4
_KNN = 32
_EPS = 1e-5


def _knn_body(scoorT_ref, coor_ref, oidx_ref, d_ref, *, mt, n):
    b = pl.program_id(0)
    c = coor_ref[0]
    c0 = c[:, 0:1]
    c1 = c[:, 1:2]
    c2 = c[:, 2:3]
    s = scoorT_ref[0]
    s0 = s[0:1, :]
    s1 = s[1:2, :]
    s2 = s[2:3, :]
    dot = jnp.dot(c, s, preferred_element_type=jnp.float32)
    cnorm = c0 * c0 + c1 * c1 + c2 * c2
    snorm = s0 * s0 + s1 * s1 + s2 * s2
    d_ref[...] = (snorm + cnorm) - 2.0 * dot
    iota = lax.broadcasted_iota(jnp.int32, (n, mt), 0)
    base = b * n
    nq = n // 8

    def body(j, carry):
        d = d_ref[...]
        d3 = d.reshape(nq, 8, mt)
        m = jnp.min(jnp.min(d3, axis=0), axis=0, keepdims=True)
        cand = jnp.where(d3 <= m[None], iota.reshape(nq, 8, mt), jnp.int32(n))
        idx = jnp.min(jnp.min(cand, axis=0), axis=0, keepdims=True)
        oidx_ref[0, pl.ds(j, 1), :] = idx + base
        d_ref[...] = jnp.where(iota == idx, jnp.float32(jnp.inf), d)
        return carry

    lax.fori_loop(0, _KNN, body, 0)


def _knn_topk(scoorT, coor, interpret=False):
    bsz, _, m = scoorT.shape
    n = coor.shape[1]
    mt = 256
    body = functools.partial(_knn_body, mt=mt, n=n)
    return pl.pallas_call(
        body,
        grid=(bsz, m // mt),
        in_specs=[
            pl.BlockSpec((1, 8, mt), lambda b, i: (b, 0, i)),
            pl.BlockSpec((1, n, 8), lambda b, i: (b, 0, 0)),
        ],
        out_specs=pl.BlockSpec((1, _KNN, mt), lambda b, i: (b, 0, i)),
        out_shape=jax.ShapeDtypeStruct((bsz, _KNN, m), jnp.int32),
        scratch_shapes=[pltpu.VMEM((n, mt), jnp.float32)],
        interpret=interpret,
    )(scoorT, coor)


def _gather_rows(xflat, gidx3):
    nw, nch, p = gidx3.shape
    c = xflat.shape[1]
    per_w = nch * p
    info = plsc.get_sparse_core_info()
    ncores = info.num_cores
    mesh = plsc.VectorSubcoreMesh(core_axis_name="c", subcore_axis_name="s")

    @functools.partial(
        pl.kernel,
        mesh=mesh,
        compiler_params=pltpu.CompilerParams(use_tc_tiling_on_sc=True),
        out_type=jax.ShapeDtypeStruct((nw * per_w, c), jnp.float32),
        scratch_types=[
            pltpu.VMEM((p,), jnp.int32),
            pltpu.VMEM((p,), jnp.int32),
            pltpu.VMEM((p, c), jnp.float32),
            pltpu.VMEM((p, c), jnp.float32),
            pltpu.SemaphoreType.DMA,
            pltpu.SemaphoreType.DMA,
        ],
    )
    def k(x_hbm, idx_hbm, out_hbm, idx0, idx1, rows0, rows1, sem0, sem1):
        wid = lax.axis_index("s") * ncores + lax.axis_index("c")
        base = wid * per_w

        def pair(i, carry):
            j0 = 2 * i
            j1 = 2 * i + 1
            pltpu.sync_copy(idx_hbm.at[wid, j0], idx0)
            cp0 = pltpu.async_copy(x_hbm.at[idx0], rows0, sem0)
            pltpu.sync_copy(idx_hbm.at[wid, j1], idx1)
            cp1 = pltpu.async_copy(x_hbm.at[idx1], rows1, sem1)
            cp0.wait()
            pltpu.sync_copy(rows0, out_hbm.at[pl.ds(base + j0 * p, p)])
            cp1.wait()
            pltpu.sync_copy(rows1, out_hbm.at[pl.ds(base + j1 * p, p)])
            return carry

        lax.fori_loop(0, nch // 2, pair, 0)

    return k(xflat, gidx3)


def _l1(g_ref, sx_ref, w1b_ref, w1d_ref, b1_ref, *, mt, oc):
    c = sx_ref.shape[2]
    u = jnp.dot(g_ref[0].reshape(_KNN * mt, c), w1b_ref[...],
                preferred_element_type=jnp.float32)
    v = jnp.dot(sx_ref[0], w1d_ref[...], preferred_element_type=jnp.float32)
    v = v + b1_ref[...]
    return (u.reshape(_KNN, mt, oc) + v[None, :, :]).reshape(_KNN * mt, oc)


def _acc_sums(sums_ref, h):
    ssum = jnp.sum(h, axis=0, keepdims=True)
    ssq = jnp.sum(h * h, axis=0, keepdims=True)
    part = jnp.concatenate([ssum, ssq], axis=0)
    first = (pl.program_id(0) == 0) & (pl.program_id(1) == 0)

    @pl.when(first)
    def _():
        sums_ref[...] = part

    @pl.when(jnp.logical_not(first))
    def _():
        sums_ref[...] = sums_ref[...] + part


def _stats1_body(g_ref, sx_ref, w1b_ref, w1d_ref, b1_ref, sums_ref, *, mt, oc):
    h = _l1(g_ref, sx_ref, w1b_ref, w1d_ref, b1_ref, mt=mt, oc=oc)
    _acc_sums(sums_ref, h)


def _stats1(gathered4, sx, w1bT, w1dT, b1r, interpret=False):
    bsz, _, m, c = gathered4.shape
    oc = w1bT.shape[1]
    mt = 64
    body = functools.partial(_stats1_body, mt=mt, oc=oc)
    return pl.pallas_call(
        body,
        grid=(bsz, m // mt),
        in_specs=[
            pl.BlockSpec((1, _KNN, mt, c), lambda b, i: (b, 0, i, 0)),
            pl.BlockSpec((1, mt, c), lambda b, i: (b, i, 0)),
            pl.BlockSpec((c, oc), lambda b, i: (0, 0)),
            pl.BlockSpec((c, oc), lambda b, i: (0, 0)),
            pl.BlockSpec((1, oc), lambda b, i: (0, 0)),
        ],
        out_specs=pl.BlockSpec((2, oc), lambda b, i: (0, 0)),
        out_shape=jax.ShapeDtypeStruct((2, oc), jnp.float32),
        interpret=interpret,
    )(gathered4, sx, w1bT, w1dT, b1r)


def _layer2_body(g_ref, sx_ref, w1b_ref, w1d_ref, b1_ref, sums1_ref, g1_ref,
                 be1_ref, w2_ref, b2_ref, hmax_ref, hmin_ref, sums2_ref, *,
                 mt, oc, tn):
    mean = sums1_ref[0:1, :] * (1.0 / tn)
    ex2 = sums1_ref[1:2, :] * (1.0 / tn)
    var = ex2 - mean * mean
    s1 = g1_ref[...] / jnp.sqrt(var + _EPS)
    t1 = be1_ref[...] - mean * s1

    h = _l1(g_ref, sx_ref, w1b_ref, w1d_ref, b1_ref, mt=mt, oc=oc)
    act = jnp.maximum(h * s1 + t1, 0.0)
    h2 = jnp.dot(act, w2_ref[...], preferred_element_type=jnp.float32)
    h2 = h2 + b2_ref[...]
    h23 = h2.reshape(_KNN, mt, oc)
    hmax_ref[0] = jnp.max(h23, axis=0)
    hmin_ref[0] = jnp.min(h23, axis=0)
    _acc_sums(sums2_ref, h2)


def _layer2(gathered4, sx, w1bT, w1dT, b1r, sums1, g1r, be1r, w2T, b2r,
            interpret=False):
    bsz, _, m, c = gathered4.shape
    oc = w2T.shape[1]
    mt = 64
    tn = float(bsz * m * _KNN)
    body = functools.partial(_layer2_body, mt=mt, oc=oc, tn=tn)
    return pl.pallas_call(
        body,
        grid=(bsz, m // mt),
        in_specs=[
            pl.BlockSpec((1, _KNN, mt, c), lambda b, i: (b, 0, i, 0)),
            pl.BlockSpec((1, mt, c), lambda b, i: (b, i, 0)),
            pl.BlockSpec((c, oc), lambda b, i: (0, 0)),
            pl.BlockSpec((c, oc), lambda b, i: (0, 0)),
            pl.BlockSpec((1, oc), lambda b, i: (0, 0)),
            pl.BlockSpec((2, oc), lambda b, i: (0, 0)),
            pl.BlockSpec((1, oc), lambda b, i: (0, 0)),
            pl.BlockSpec((1, oc), lambda b, i: (0, 0)),
            pl.BlockSpec((oc, oc), lambda b, i: (0, 0)),
            pl.BlockSpec((1, oc), lambda b, i: (0, 0)),
        ],
        out_specs=[
            pl.BlockSpec((1, mt, oc), lambda b, i: (b, i, 0)),
            pl.BlockSpec((1, mt, oc), lambda b, i: (b, i, 0)),
            pl.BlockSpec((2, oc), lambda b, i: (0, 0)),
        ],
        out_shape=[
            jax.ShapeDtypeStruct((bsz, m, oc), jnp.float32),
            jax.ShapeDtypeStruct((bsz, m, oc), jnp.float32),
            jax.ShapeDtypeStruct((2, oc), jnp.float32),
        ],
        interpret=interpret,
    )(gathered4, sx, w1bT, w1dT, b1r, sums1, g1r, be1r, w2T, b2r)


def _finish_body(hmax_ref, hmin_ref, sums2_ref, g2_ref, be2_ref, out_ref, *, tn):
    mean = sums2_ref[0:1, :] * (1.0 / tn)
    ex2 = sums2_ref[1:2, :] * (1.0 / tn)
    var = ex2 - mean * mean
    s2 = g2_ref[...] / jnp.sqrt(var + _EPS)
    t2 = be2_ref[...] - mean * s2
    a = jnp.where(s2 >= 0.0, hmax_ref[...], hmin_ref[...])
    out_ref[...] = jnp.maximum(a * s2 + t2, 0.0)


def _finish(hmax, hmin, sums2, g2r, be2r, interpret=False):
    t, oc = hmax.shape
    tt = 512
    tn = float(t * _KNN)
    body = functools.partial(_finish_body, tn=tn)
    return pl.pallas_call(
        body,
        grid=(t // tt,),
        in_specs=[
            pl.BlockSpec((tt, oc), lambda i: (i, 0)),
            pl.BlockSpec((tt, oc), lambda i: (i, 0)),
            pl.BlockSpec((2, oc), lambda i: (0, 0)),
            pl.BlockSpec((1, oc), lambda i: (0, 0)),
            pl.BlockSpec((1, oc), lambda i: (0, 0)),
        ],
        out_specs=pl.BlockSpec((tt, oc), lambda i: (i, 0)),
        out_shape=jax.ShapeDtypeStruct((t, oc), jnp.float32),
        interpret=interpret,
    )(hmax, hmin, sums2, g2r, be2r)


def kernel(x, coor, W1, b1, g1, be1, W2, b2, g2, be2):
    B, N, C = x.shape
    M = N // _ODR
    OC = W1.shape[0]

    indx = jax.random.permutation(jax.random.key(42), N)[:M]
    sampled_coor = coor[:, indx, :]
    sampled_x = x[:, indx, :]

    scoorT = jnp.pad(jnp.transpose(sampled_coor, (0, 2, 1)),
                     ((0, 0), (0, 5), (0, 0)))
    coor8 = jnp.pad(coor, ((0, 0), (0, 0), (0, 5)))
    gidx = _knn_topk(scoorT, coor8)

    nw = 32
    p = 256
    nch = (B * M * _KNN) // (nw * p)
    gidx3 = gidx.reshape(nw, nch, p)
    xflat = x.reshape(B * N, C)
    gathered = _gather_rows(xflat, gidx3)
    gathered4 = gathered.reshape(B, _KNN, M, C)

    w1bT = jnp.transpose(W1[:, C:])
    w1dT = jnp.transpose(W1[:, :C] - W1[:, C:])
    b1r = b1.reshape(1, OC)
    sums1 = _stats1(gathered4, sampled_x, w1bT, w1dT, b1r)

    w2T = jnp.transpose(W2)
    hmax, hmin, sums2 = _layer2(
        gathered4, sampled_x, w1bT, w1dT, b1r, sums1,
        g1.reshape(1, OC), be1.reshape(1, OC), w2T, b2.reshape(1, OC))

    out = _finish(hmax.reshape(B * M, OC), hmin.reshape(B * M, OC), sums2,
                  g2.reshape(1, OC), be2.reshape(1, OC))
    return (out.reshape(B, M, OC), sampled_coor)


# speedup vs baseline: 1.2343x; 1.2343x over previous
import functools

import jax
import jax.numpy as jnp
from jax import lax
from jax.experimental import pallas as pl
from jax.experimental.pallas import tpu as pltpu
from jax.experimental.pallas import tpu_sc as plsc

_ODR = 4
_KNN = 32
_EPS = 1e-5


def _knn_body(scoorT_ref, coor_ref, oidx_ref, d_ref, *, mt, n):
    b = pl.program_id(0)
    c = coor_ref[0]
    c0 = c[:, 0:1]
    c1 = c[:, 1:2]
    c2 = c[:, 2:3]
    s = scoorT_ref[0]
    s0 = s[0:1, :]
    s1 = s[1:2, :]
    s2 = s[2:3, :]
    dot = jnp.dot(c, s, preferred_element_type=jnp.float32)
    cnorm = c0 * c0 + c1 * c1 + c2 * c2
    snorm = s0 * s0 + s1 * s1 + s2 * s2
    d0 = (snorm + cnorm) - 2.0 * dot
    d_ref[...] = d0
    iota = lax.broadcasted_iota(jnp.int32, (n, mt), 0)
    base = b * n
    nq = n // 8

    def _rmin(a):
        a3 = a.reshape(nq, 8, mt)
        return jnp.min(jnp.min(a3, axis=0), axis=0, keepdims=True)

    def body(j, m):
        d = d_ref[...]
        cand = jnp.where(d <= m, iota, jnp.int32(n))
        idx = jnp.min(jnp.min(cand.reshape(nq, 8, mt), axis=0), axis=0,
                      keepdims=True)
        oidx_ref[0, pl.ds(j, 1), :] = idx + base
        dn = jnp.where(iota == idx, jnp.float32(jnp.inf), d)
        d_ref[...] = dn
        return _rmin(dn)

    lax.fori_loop(0, _KNN, body, _rmin(d0))


def _knn_topk(scoorT, coor, interpret=False):
    bsz, _, m = scoorT.shape
    n = coor.shape[1]
    mt = 256
    body = functools.partial(_knn_body, mt=mt, n=n)
    return pl.pallas_call(
        body,
        grid=(bsz, m // mt),
        in_specs=[
            pl.BlockSpec((1, 8, mt), lambda b, i: (b, 0, i)),
            pl.BlockSpec((1, n, 8), lambda b, i: (b, 0, 0)),
        ],
        out_specs=pl.BlockSpec((1, _KNN, mt), lambda b, i: (b, 0, i)),
        out_shape=jax.ShapeDtypeStruct((bsz, _KNN, m), jnp.int32),
        scratch_shapes=[pltpu.VMEM((n, mt), jnp.float32)],
        interpret=interpret,
    )(scoorT, coor)


def _gather_rows(xflat, gidx3):
    nw, nch, p = gidx3.shape
    c = xflat.shape[1]
    per_w = nch * p
    info = plsc.get_sparse_core_info()
    ncores = info.num_cores
    mesh = plsc.VectorSubcoreMesh(core_axis_name="c", subcore_axis_name="s")

    @functools.partial(
        pl.kernel,
        mesh=mesh,
        compiler_params=pltpu.CompilerParams(use_tc_tiling_on_sc=True),
        out_type=jax.ShapeDtypeStruct((nw * per_w, c), jnp.float32),
        scratch_types=[
            pltpu.VMEM((p,), jnp.int32),
            pltpu.VMEM((p,), jnp.int32),
            pltpu.VMEM((p, c), jnp.float32),
            pltpu.VMEM((p, c), jnp.float32),
            pltpu.SemaphoreType.DMA,
            pltpu.SemaphoreType.DMA,
        ],
    )
    def k(x_hbm, idx_hbm, out_hbm, idx0, idx1, rows0, rows1, sem0, sem1):
        wid = lax.axis_index("s") * ncores + lax.axis_index("c")
        base = wid * per_w

        def pair(i, carry):
            j0 = 2 * i
            j1 = 2 * i + 1
            pltpu.sync_copy(idx_hbm.at[wid, j0], idx0)
            cp0 = pltpu.async_copy(x_hbm.at[idx0], rows0, sem0)
            pltpu.sync_copy(idx_hbm.at[wid, j1], idx1)
            cp1 = pltpu.async_copy(x_hbm.at[idx1], rows1, sem1)
            cp0.wait()
            pltpu.sync_copy(rows0, out_hbm.at[pl.ds(base + j0 * p, p)])
            cp1.wait()
            pltpu.sync_copy(rows1, out_hbm.at[pl.ds(base + j1 * p, p)])
            return carry

        lax.fori_loop(0, nch // 2, pair, 0)

    return k(xflat, gidx3)


def _l1(g_ref, sx_ref, w1b_ref, w1d_ref, b1_ref, *, mt, oc):
    c = sx_ref.shape[2]
    u = jnp.dot(g_ref[0].reshape(_KNN * mt, c), w1b_ref[...],
                preferred_element_type=jnp.float32)
    v = jnp.dot(sx_ref[0], w1d_ref[...], preferred_element_type=jnp.float32)
    v = v + b1_ref[...]
    return (u.reshape(_KNN, mt, oc) + v[None, :, :]).reshape(_KNN * mt, oc)


def _acc_sums(sums_ref, h):
    ssum = jnp.sum(h, axis=0, keepdims=True)
    ssq = jnp.sum(h * h, axis=0, keepdims=True)
    part = jnp.concatenate([ssum, ssq], axis=0)
    first = (pl.program_id(0) == 0) & (pl.program_id(1) == 0)

    @pl.when(first)
    def _():
        sums_ref[...] = part

    @pl.when(jnp.logical_not(first))
    def _():
        sums_ref[...] = sums_ref[...] + part


def _stats1_body(g_ref, sx_ref, w1b_ref, w1d_ref, b1_ref, sums_ref, *, mt, oc):
    h = _l1(g_ref, sx_ref, w1b_ref, w1d_ref, b1_ref, mt=mt, oc=oc)
    _acc_sums(sums_ref, h)


def _stats1(gathered4, sx, w1bT, w1dT, b1r, interpret=False):
    bsz, _, m, c = gathered4.shape
    oc = w1bT.shape[1]
    mt = 64
    body = functools.partial(_stats1_body, mt=mt, oc=oc)
    return pl.pallas_call(
        body,
        grid=(bsz, m // mt),
        in_specs=[
            pl.BlockSpec((1, _KNN, mt, c), lambda b, i: (b, 0, i, 0)),
            pl.BlockSpec((1, mt, c), lambda b, i: (b, i, 0)),
            pl.BlockSpec((c, oc), lambda b, i: (0, 0)),
            pl.BlockSpec((c, oc), lambda b, i: (0, 0)),
            pl.BlockSpec((1, oc), lambda b, i: (0, 0)),
        ],
        out_specs=pl.BlockSpec((2, oc), lambda b, i: (0, 0)),
        out_shape=jax.ShapeDtypeStruct((2, oc), jnp.float32),
        interpret=interpret,
    )(gathered4, sx, w1bT, w1dT, b1r)


def _layer2_body(g_ref, sx_ref, w1b_ref, w1d_ref, b1_ref, sums1_ref, g1_ref,
                 be1_ref, w2_ref, b2_ref, hmax_ref, hmin_ref, sums2_ref, *,
                 mt, oc, tn):
    mean = sums1_ref[0:1, :] * (1.0 / tn)
    ex2 = sums1_ref[1:2, :] * (1.0 / tn)
    var = ex2 - mean * mean
    s1 = g1_ref[...] / jnp.sqrt(var + _EPS)
    t1 = be1_ref[...] - mean * s1

    h = _l1(g_ref, sx_ref, w1b_ref, w1d_ref, b1_ref, mt=mt, oc=oc)
    act = jnp.maximum(h * s1 + t1, 0.0)
    h2 = jnp.dot(act, w2_ref[...], preferred_element_type=jnp.float32)
    h2 = h2 + b2_ref[...]
    h23 = h2.reshape(_KNN, mt, oc)
    hmax_ref[0] = jnp.max(h23, axis=0)
    hmin_ref[0] = jnp.min(h23, axis=0)
    _acc_sums(sums2_ref, h2)


def _layer2(gathered4, sx, w1bT, w1dT, b1r, sums1, g1r, be1r, w2T, b2r,
            interpret=False):
    bsz, _, m, c = gathered4.shape
    oc = w2T.shape[1]
    mt = 64
    tn = float(bsz * m * _KNN)
    body = functools.partial(_layer2_body, mt=mt, oc=oc, tn=tn)
    return pl.pallas_call(
        body,
        grid=(bsz, m // mt),
        in_specs=[
            pl.BlockSpec((1, _KNN, mt, c), lambda b, i: (b, 0, i, 0)),
            pl.BlockSpec((1, mt, c), lambda b, i: (b, i, 0)),
            pl.BlockSpec((c, oc), lambda b, i: (0, 0)),
            pl.BlockSpec((c, oc), lambda b, i: (0, 0)),
            pl.BlockSpec((1, oc), lambda b, i: (0, 0)),
            pl.BlockSpec((2, oc), lambda b, i: (0, 0)),
            pl.BlockSpec((1, oc), lambda b, i: (0, 0)),
            pl.BlockSpec((1, oc), lambda b, i: (0, 0)),
            pl.BlockSpec((oc, oc), lambda b, i: (0, 0)),
            pl.BlockSpec((1, oc), lambda b, i: (0, 0)),
        ],
        out_specs=[
            pl.BlockSpec((1, mt, oc), lambda b, i: (b, i, 0)),
            pl.BlockSpec((1, mt, oc), lambda b, i: (b, i, 0)),
            pl.BlockSpec((2, oc), lambda b, i: (0, 0)),
        ],
        out_shape=[
            jax.ShapeDtypeStruct((bsz, m, oc), jnp.float32),
            jax.ShapeDtypeStruct((bsz, m, oc), jnp.float32),
            jax.ShapeDtypeStruct((2, oc), jnp.float32),
        ],
        interpret=interpret,
    )(gathered4, sx, w1bT, w1dT, b1r, sums1, g1r, be1r, w2T, b2r)


def _finish_body(hmax_ref, hmin_ref, sums2_ref, g2_ref, be2_ref, out_ref, *, tn):
    mean = sums2_ref[0:1, :] * (1.0 / tn)
    ex2 = sums2_ref[1:2, :] * (1.0 / tn)
    var = ex2 - mean * mean
    s2 = g2_ref[...] / jnp.sqrt(var + _EPS)
    t2 = be2_ref[...] - mean * s2
    a = jnp.where(s2 >= 0.0, hmax_ref[...], hmin_ref[...])
    out_ref[...] = jnp.maximum(a * s2 + t2, 0.0)


def _finish(hmax, hmin, sums2, g2r, be2r, interpret=False):
    t, oc = hmax.shape
    tt = 512
    tn = float(t * _KNN)
    body = functools.partial(_finish_body, tn=tn)
    return pl.pallas_call(
        body,
        grid=(t // tt,),
        in_specs=[
            pl.BlockSpec((tt, oc), lambda i: (i, 0)),
            pl.BlockSpec((tt, oc), lambda i: (i, 0)),
            pl.BlockSpec((2, oc), lambda i: (0, 0)),
            pl.BlockSpec((1, oc), lambda i: (0, 0)),
            pl.BlockSpec((1, oc), lambda i: (0, 0)),
        ],
        out_specs=pl.BlockSpec((tt, oc), lambda i: (i, 0)),
        out_shape=jax.ShapeDtypeStruct((t, oc), jnp.float32),
        interpret=interpret,
    )(hmax, hmin, sums2, g2r, be2r)


def kernel(x, coor, W1, b1, g1, be1, W2, b2, g2, be2):
    B, N, C = x.shape
    M = N // _ODR
    OC = W1.shape[0]

    indx = jax.random.permutation(jax.random.key(42), N)[:M]
    sampled_coor = coor[:, indx, :]
    sampled_x = x[:, indx, :]

    scoorT = jnp.pad(jnp.transpose(sampled_coor, (0, 2, 1)),
                     ((0, 0), (0, 5), (0, 0)))
    coor8 = jnp.pad(coor, ((0, 0), (0, 0), (0, 5)))
    gidx = _knn_topk(scoorT, coor8)

    nw = 32
    p = 256
    nch = (B * M * _KNN) // (nw * p)
    gidx3 = gidx.reshape(nw, nch, p)
    xflat = x.reshape(B * N, C)
    gathered = _gather_rows(xflat, gidx3)
    gathered4 = gathered.reshape(B, _KNN, M, C)

    w1bT = jnp.transpose(W1[:, C:])
    w1dT = jnp.transpose(W1[:, :C] - W1[:, C:])
    b1r = b1.reshape(1, OC)
    sums1 = _stats1(gathered4, sampled_x, w1bT, w1dT, b1r)

    w2T = jnp.transpose(W2)
    hmax, hmin, sums2 = _layer2(
        gathered4, sampled_x, w1bT, w1dT, b1r, sums1,
        g1.reshape(1, OC), be1.reshape(1, OC), w2T, b2.reshape(1, OC))

    out = _finish(hmax.reshape(B * M, OC), hmin.reshape(B * M, OC), sums2,
                  g2.reshape(1, OC), be2.reshape(1, OC))
    return (out.reshape(B, M, OC), sampled_coor)


# speedup vs baseline: 1.2641x; 1.0242x over previous
import functools

import jax
import jax.numpy as jnp
from jax import lax
from jax.experimental import pallas as pl
from jax.experimental.pallas import tpu as pltpu
from jax.experimental.pallas import tpu_sc as plsc

_ODR = 4
_KNN = 32
_EPS = 1e-5


def _knn_body(scoorT_ref, coor_ref, oidx_ref, d_ref, *, mt, n):
    b = pl.program_id(0)
    c = coor_ref[0]
    c0 = c[:, 0:1]
    c1 = c[:, 1:2]
    c2 = c[:, 2:3]
    s = scoorT_ref[0]
    s0 = s[0:1, :]
    s1 = s[1:2, :]
    s2 = s[2:3, :]
    dot = jnp.dot(c, s, preferred_element_type=jnp.float32)
    cnorm = c0 * c0 + c1 * c1 + c2 * c2
    snorm = s0 * s0 + s1 * s1 + s2 * s2
    d0 = (snorm + cnorm) - 2.0 * dot
    d_ref[...] = d0
    iota = lax.broadcasted_iota(jnp.int32, (n, mt), 0)
    base = b * n
    nq = n // 8

    def _rmin(a):
        a3 = a.reshape(nq, 8, mt)
        return jnp.min(jnp.min(a3, axis=0), axis=0, keepdims=True)

    def body(j, m):
        d = d_ref[...]
        cand = jnp.where(d <= m, iota, jnp.int32(n))
        idx = jnp.min(jnp.min(cand.reshape(nq, 8, mt), axis=0), axis=0,
                      keepdims=True)
        oidx_ref[0, pl.ds(j, 1), :] = idx + base
        dn = jnp.where(iota == idx, jnp.float32(jnp.inf), d)
        d_ref[...] = dn
        return _rmin(dn)

    lax.fori_loop(0, _KNN, body, _rmin(d0))


def _knn_topk(scoorT, coor, interpret=False):
    bsz, _, m = scoorT.shape
    n = coor.shape[1]
    mt = 512
    body = functools.partial(_knn_body, mt=mt, n=n)
    return pl.pallas_call(
        body,
        grid=(bsz, m // mt),
        in_specs=[
            pl.BlockSpec((1, 8, mt), lambda b, i: (b, 0, i)),
            pl.BlockSpec((1, n, 8), lambda b, i: (b, 0, 0)),
        ],
        out_specs=pl.BlockSpec((1, _KNN, mt), lambda b, i: (b, 0, i)),
        out_shape=jax.ShapeDtypeStruct((bsz, _KNN, m), jnp.int32),
        scratch_shapes=[pltpu.VMEM((n, mt), jnp.float32)],
        interpret=interpret,
    )(scoorT, coor)


def _gather_rows(xflat, gidx3):
    nw, nch, p = gidx3.shape
    c = xflat.shape[1]
    per_w = nch * p
    info = plsc.get_sparse_core_info()
    ncores = info.num_cores
    mesh = plsc.VectorSubcoreMesh(core_axis_name="c", subcore_axis_name="s")

    @functools.partial(
        pl.kernel,
        mesh=mesh,
        compiler_params=pltpu.CompilerParams(use_tc_tiling_on_sc=True),
        out_type=jax.ShapeDtypeStruct((nw * per_w, c), jnp.float32),
        scratch_types=[
            pltpu.VMEM((p,), jnp.int32),
            pltpu.VMEM((p,), jnp.int32),
            pltpu.VMEM((p, c), jnp.float32),
            pltpu.VMEM((p, c), jnp.float32),
            pltpu.SemaphoreType.DMA,
            pltpu.SemaphoreType.DMA,
        ],
    )
    def k(x_hbm, idx_hbm, out_hbm, idx0, idx1, rows0, rows1, sem0, sem1):
        wid = lax.axis_index("s") * ncores + lax.axis_index("c")
        base = wid * per_w

        def pair(i, carry):
            j0 = 2 * i
            j1 = 2 * i + 1
            pltpu.sync_copy(idx_hbm.at[wid, j0], idx0)
            cp0 = pltpu.async_copy(x_hbm.at[idx0], rows0, sem0)
            pltpu.sync_copy(idx_hbm.at[wid, j1], idx1)
            cp1 = pltpu.async_copy(x_hbm.at[idx1], rows1, sem1)
            cp0.wait()
            pltpu.sync_copy(rows0, out_hbm.at[pl.ds(base + j0 * p, p)])
            cp1.wait()
            pltpu.sync_copy(rows1, out_hbm.at[pl.ds(base + j1 * p, p)])
            return carry

        lax.fori_loop(0, nch // 2, pair, 0)

    return k(xflat, gidx3)


def _l1(g_ref, sx_ref, w1b_ref, w1d_ref, b1_ref, *, mt, oc):
    c = sx_ref.shape[2]
    u = jnp.dot(g_ref[0].reshape(_KNN * mt, c), w1b_ref[...],
                preferred_element_type=jnp.float32)
    v = jnp.dot(sx_ref[0], w1d_ref[...], preferred_element_type=jnp.float32)
    v = v + b1_ref[...]
    return (u.reshape(_KNN, mt, oc) + v[None, :, :]).reshape(_KNN * mt, oc)


def _acc_sums(sums_ref, h):
    ssum = jnp.sum(h, axis=0, keepdims=True)
    ssq = jnp.sum(h * h, axis=0, keepdims=True)
    part = jnp.concatenate([ssum, ssq], axis=0)
    first = (pl.program_id(0) == 0) & (pl.program_id(1) == 0)

    @pl.when(first)
    def _():
        sums_ref[...] = part

    @pl.when(jnp.logical_not(first))
    def _():
        sums_ref[...] = sums_ref[...] + part


def _stats1_body(g_ref, sx_ref, w1b_ref, w1d_ref, b1_ref, sums_ref, *, mt, oc):
    h = _l1(g_ref, sx_ref, w1b_ref, w1d_ref, b1_ref, mt=mt, oc=oc)
    _acc_sums(sums_ref, h)


def _stats1(gathered4, sx, w1bT, w1dT, b1r, interpret=False):
    bsz, _, m, c = gathered4.shape
    oc = w1bT.shape[1]
    mt = 64
    body = functools.partial(_stats1_body, mt=mt, oc=oc)
    return pl.pallas_call(
        body,
        grid=(bsz, m // mt),
        in_specs=[
            pl.BlockSpec((1, _KNN, mt, c), lambda b, i: (b, 0, i, 0)),
            pl.BlockSpec((1, mt, c), lambda b, i: (b, i, 0)),
            pl.BlockSpec((c, oc), lambda b, i: (0, 0)),
            pl.BlockSpec((c, oc), lambda b, i: (0, 0)),
            pl.BlockSpec((1, oc), lambda b, i: (0, 0)),
        ],
        out_specs=pl.BlockSpec((2, oc), lambda b, i: (0, 0)),
        out_shape=jax.ShapeDtypeStruct((2, oc), jnp.float32),
        interpret=interpret,
    )(gathered4, sx, w1bT, w1dT, b1r)


def _layer2_body(g_ref, sx_ref, w1b_ref, w1d_ref, b1_ref, sums1_ref, g1_ref,
                 be1_ref, w2_ref, b2_ref, hmax_ref, hmin_ref, sums2_ref, *,
                 mt, oc, tn):
    mean = sums1_ref[0:1, :] * (1.0 / tn)
    ex2 = sums1_ref[1:2, :] * (1.0 / tn)
    var = ex2 - mean * mean
    s1 = g1_ref[...] / jnp.sqrt(var + _EPS)
    t1 = be1_ref[...] - mean * s1

    h = _l1(g_ref, sx_ref, w1b_ref, w1d_ref, b1_ref, mt=mt, oc=oc)
    act = jnp.maximum(h * s1 + t1, 0.0)
    h2 = jnp.dot(act, w2_ref[...], preferred_element_type=jnp.float32)
    h2 = h2 + b2_ref[...]
    h23 = h2.reshape(_KNN, mt, oc)
    hmax_ref[0] = jnp.max(h23, axis=0)
    hmin_ref[0] = jnp.min(h23, axis=0)
    _acc_sums(sums2_ref, h2)


def _layer2(gathered4, sx, w1bT, w1dT, b1r, sums1, g1r, be1r, w2T, b2r,
            interpret=False):
    bsz, _, m, c = gathered4.shape
    oc = w2T.shape[1]
    mt = 64
    tn = float(bsz * m * _KNN)
    body = functools.partial(_layer2_body, mt=mt, oc=oc, tn=tn)
    return pl.pallas_call(
        body,
        grid=(bsz, m // mt),
        in_specs=[
            pl.BlockSpec((1, _KNN, mt, c), lambda b, i: (b, 0, i, 0)),
            pl.BlockSpec((1, mt, c), lambda b, i: (b, i, 0)),
            pl.BlockSpec((c, oc), lambda b, i: (0, 0)),
            pl.BlockSpec((c, oc), lambda b, i: (0, 0)),
            pl.BlockSpec((1, oc), lambda b, i: (0, 0)),
            pl.BlockSpec((2, oc), lambda b, i: (0, 0)),
            pl.BlockSpec((1, oc), lambda b, i: (0, 0)),
            pl.BlockSpec((1, oc), lambda b, i: (0, 0)),
            pl.BlockSpec((oc, oc), lambda b, i: (0, 0)),
            pl.BlockSpec((1, oc), lambda b, i: (0, 0)),
        ],
        out_specs=[
            pl.BlockSpec((1, mt, oc), lambda b, i: (b, i, 0)),
            pl.BlockSpec((1, mt, oc), lambda b, i: (b, i, 0)),
            pl.BlockSpec((2, oc), lambda b, i: (0, 0)),
        ],
        out_shape=[
            jax.ShapeDtypeStruct((bsz, m, oc), jnp.float32),
            jax.ShapeDtypeStruct((bsz, m, oc), jnp.float32),
            jax.ShapeDtypeStruct((2, oc), jnp.float32),
        ],
        interpret=interpret,
    )(gathered4, sx, w1bT, w1dT, b1r, sums1, g1r, be1r, w2T, b2r)


def _finish_body(hmax_ref, hmin_ref, sums2_ref, g2_ref, be2_ref, out_ref, *, tn):
    mean = sums2_ref[0:1, :] * (1.0 / tn)
    ex2 = sums2_ref[1:2, :] * (1.0 / tn)
    var = ex2 - mean * mean
    s2 = g2_ref[...] / jnp.sqrt(var + _EPS)
    t2 = be2_ref[...] - mean * s2
    a = jnp.where(s2 >= 0.0, hmax_ref[...], hmin_ref[...])
    out_ref[...] = jnp.maximum(a * s2 + t2, 0.0)


def _finish(hmax, hmin, sums2, g2r, be2r, interpret=False):
    t, oc = hmax.shape
    tt = 512
    tn = float(t * _KNN)
    body = functools.partial(_finish_body, tn=tn)
    return pl.pallas_call(
        body,
        grid=(t // tt,),
        in_specs=[
            pl.BlockSpec((tt, oc), lambda i: (i, 0)),
            pl.BlockSpec((tt, oc), lambda i: (i, 0)),
            pl.BlockSpec((2, oc), lambda i: (0, 0)),
            pl.BlockSpec((1, oc), lambda i: (0, 0)),
            pl.BlockSpec((1, oc), lambda i: (0, 0)),
        ],
        out_specs=pl.BlockSpec((tt, oc), lambda i: (i, 0)),
        out_shape=jax.ShapeDtypeStruct((t, oc), jnp.float32),
        interpret=interpret,
    )(hmax, hmin, sums2, g2r, be2r)


def kernel(x, coor, W1, b1, g1, be1, W2, b2, g2, be2):
    B, N, C = x.shape
    M = N // _ODR
    OC = W1.shape[0]

    indx = jax.random.permutation(jax.random.key(42), N)[:M]
    sampled_coor = coor[:, indx, :]
    sampled_x = x[:, indx, :]

    scoorT = jnp.pad(jnp.transpose(sampled_coor, (0, 2, 1)),
                     ((0, 0), (0, 5), (0, 0)))
    coor8 = jnp.pad(coor, ((0, 0), (0, 0), (0, 5)))
    gidx = _knn_topk(scoorT, coor8)

    nw = 32
    p = 256
    nch = (B * M * _KNN) // (nw * p)
    gidx3 = gidx.reshape(nw, nch, p)
    xflat = x.reshape(B * N, C)
    gathered = _gather_rows(xflat, gidx3)
    gathered4 = gathered.reshape(B, _KNN, M, C)

    w1bT = jnp.transpose(W1[:, C:])
    w1dT = jnp.transpose(W1[:, :C] - W1[:, C:])
    b1r = b1.reshape(1, OC)
    sums1 = _stats1(gathered4, sampled_x, w1bT, w1dT, b1r)

    w2T = jnp.transpose(W2)
    hmax, hmin, sums2 = _layer2(
        gathered4, sampled_x, w1bT, w1dT, b1r, sums1,
        g1.reshape(1, OC), be1.reshape(1, OC), w2T, b2.reshape(1, OC))

    out = _finish(hmax.reshape(B * M, OC), hmin.reshape(B * M, OC), sums2,
                  g2.reshape(1, OC), be2.reshape(1, OC))
    return (out.reshape(B, M, OC), sampled_coor)
